# baseline (speedup 1.0000x reference)
import functools

import jax
import jax.numpy as jnp
from jax import lax
from jax.experimental import pallas as pl
from jax.experimental.pallas import tpu as pltpu
from jax.experimental.pallas import tpu_sc as plsc

B, L = 1024, 200
NUCL_DIM, SPEC_DIM = 128, 64
OUT_DIM = NUCL_DIM + 2 * SPEC_DIM
NW = 32
BPW = B // NW
NVR = OUT_DIM // 16
FULL_GROUPS, TAIL = L // 16, L % 16


def _body(seq_ref, pbsf_ref, rtf_ref, nucl_ref, pbst_ref, rtt_ref,
          out_ref,
          nucl_v, pbst_v, rtt_v, ctab_v, pbsf_v, rtf_v, seq_all,
          rb0, rb1, so0, so1):
    wid = lax.axis_index("s") * 2 + lax.axis_index("c")
    base = wid * BPW

    pltpu.sync_copy(nucl_ref, nucl_v)
    pltpu.sync_copy(pbst_ref, pbst_v)
    pltpu.sync_copy(rtt_ref, rtt_v)
    for pi in range(2):
        for ri in range(2):
            for v in range(6):
                row = 12 * pi + 6 * ri + v
                for k in range(NUCL_DIM // 16):
                    ctab_v[row, pl.ds(16 * k, 16)] = nucl_v[v, pl.ds(16 * k, 16)]
                for k in range(SPEC_DIM // 16):
                    ctab_v[row, pl.ds(NUCL_DIM + 16 * k, 16)] = pbst_v[pi, pl.ds(16 * k, 16)]
                for k in range(SPEC_DIM // 16):
                    ctab_v[row, pl.ds(NUCL_DIM + SPEC_DIM + 16 * k, 16)] = rtt_v[ri, pl.ds(16 * k, 16)]

    pltpu.sync_copy(pbsf_ref.at[pl.ds(base, BPW)], pbsf_v)
    pltpu.sync_copy(rtf_ref.at[pl.ds(base, BPW)], rtf_v)
    half = jnp.full((16,), 0.5, jnp.float32)
    combos = []
    for k in range(BPW // 16):
        pv = pbsf_v[pl.ds(16 * k, 16)]
        rv = rtf_v[pl.ds(16 * k, 16)]
        combo = jnp.where(pv > half, jnp.int32(12), jnp.int32(0))
        combos.append(combo + jnp.where(rv > half, jnp.int32(6), jnp.int32(0)))
    lane_ids = lax.iota(jnp.int32, 16)

    pltpu.sync_copy(seq_ref.at[pl.ds(base * L, BPW * L)], seq_all.at[pl.ds(0, BPW * L)])

    def build_rows(j, rb):
        in_lo = jnp.full((16,), j < 16)
        cvec = jnp.where(in_lo, combos[0], combos[1])
        off = jnp.sum(jnp.where(lane_ids == (j % 16), cvec, jnp.int32(0)))

        def do_rows(grp_base, rowvec, n_rows):
            for dr in range(n_rows):
                r = jnp.sum(jnp.where(lane_ids == dr, rowvec, jnp.int32(0)))
                for k in range(NVR):
                    rb[grp_base + dr, pl.ds(16 * k, 16)] = ctab_v[r, pl.ds(16 * k, 16)]

        def grp(g, carry):
            rowvec = seq_all[pl.ds(j * L + 16 * g, 16)] + off
            do_rows(16 * g, rowvec, 16)
            return carry

        lax.fori_loop(0, FULL_GROUPS, grp, 0)
        tailvec = seq_all[pl.ds(j * L + 16 * FULL_GROUPS, 16)] + off
        do_rows(16 * FULL_GROUPS, tailvec, TAIL)

    rbs, sos = (rb0, rb1), (so0, so1)

    def pair(p, carry):
        for parity in range(2):
            j = 2 * p + parity
            rb, so = rbs[parity], sos[parity]

            @pl.when(p >= 1)
            def _wait_prev():
                pltpu.make_async_copy(
                    rb, out_ref.at[pl.ds(base * L, L)], so).wait()

            build_rows(j, rb)
            pltpu.async_copy(rb, out_ref.at[pl.ds((base + j) * L, L)], so)
        return carry

    lax.fori_loop(0, BPW // 2, pair, 0)
    pltpu.make_async_copy(rb0, out_ref.at[pl.ds(base * L, L)], so0).wait()
    pltpu.make_async_copy(rb1, out_ref.at[pl.ds(base * L, L)], so1).wait()


def kernel(seq, pbs_feat, rt_feat, nucl_table, pbs_table, rt_table):
    mesh = plsc.VectorSubcoreMesh(core_axis_name="c", subcore_axis_name="s")
    run = functools.partial(
        pl.kernel,
        mesh=mesh,
        compiler_params=pltpu.CompilerParams(needs_layout_passes=False),
        out_type=jax.ShapeDtypeStruct((B * L, OUT_DIM), jnp.float32),
        scratch_types=[
            pltpu.VMEM((6, NUCL_DIM), jnp.float32),
            pltpu.VMEM((2, SPEC_DIM), jnp.float32),
            pltpu.VMEM((2, SPEC_DIM), jnp.float32),
            pltpu.VMEM((24, OUT_DIM), jnp.float32),
            pltpu.VMEM((BPW,), jnp.float32),
            pltpu.VMEM((BPW,), jnp.float32),
            pltpu.VMEM((BPW * L + 16,), jnp.int32),
            pltpu.VMEM((L, OUT_DIM), jnp.float32),
            pltpu.VMEM((L, OUT_DIM), jnp.float32),
            pltpu.SemaphoreType.DMA,
            pltpu.SemaphoreType.DMA,
        ],
    )(_body)
    out = run(seq.reshape(B * L), pbs_feat, rt_feat,
              nucl_table, pbs_table, rt_table)
    return out.reshape(B, L, OUT_DIM)
